# baseline (speedup 1.0000x reference)
import jax
import jax.numpy as jnp
from jax import lax
from jax.experimental import pallas as pl
from jax.experimental.pallas import tpu as pltpu
from jax.experimental.pallas import tpu_sc as plsc

N_NODES = 100000
N_EDGES = 3200000
FEAT = 32
NPAD = 102400
N_SUBCORES = 16
SLICE = NPAD // N_SUBCORES
EDGES_PER_TILE = N_EDGES // N_SUBCORES
CH = 4000
NCH = EDGES_PER_TILE // CH
L = 16
R_ROUNDS = 16
CB = NPAD // R_ROUNDS
STRIP = CB // N_SUBCORES
U = 10


def _hist_body(eflat_hbm, deg_in_hbm, deg_out_hbm,
               hist_v, ebuf_v, acc_v, tmp_v, shared_sp,
               sem_a, sem_b, sem_r):
    c = lax.axis_index("c")
    s = lax.axis_index("s")

    zeros = jnp.zeros((L,), jnp.int32)
    ones = jnp.ones((L,), jnp.int32)

    def _zero_hist(i, carry):
        for u in range(8):
            hist_v[pl.ds(i * 8 * L + u * L, L)] = zeros
        return carry

    lax.fori_loop(0, NPAD // (8 * L), _zero_hist, 0)

    base = (1 - c) * N_EDGES + s * EDGES_PER_TILE

    def _edge_copy(k, b):
        sem = sem_a if b == 0 else sem_b
        return pltpu.make_async_copy(
            eflat_hbm.at[pl.ds(base + k * CH, CH)],
            ebuf_v.at[pl.ds(b * CH, CH)], sem)

    _edge_copy(0, 0).start()

    def _chunk_pair(kk, carry):
        for b in range(2):
            k = kk * 2 + b
            _edge_copy(k, b).wait()

            @pl.when(k + 1 < NCH)
            def _():
                _edge_copy(k + 1, 1 - b).start()

            def _scat(j, carry2):
                for u in range(U):
                    idx = ebuf_v[pl.ds(b * CH + j * (U * L) + u * L, L)]
                    plsc.addupdate_scatter(hist_v, [idx], ones)
                return carry2

            lax.fori_loop(0, CH // (U * L), _scat, 0)
        return carry

    lax.fori_loop(0, NCH // 2, _chunk_pair, 0)

    def _zero_acc(i, carry):
        acc_v[pl.ds(i * L, L)] = zeros
        return carry

    lax.fori_loop(0, SLICE // L, _zero_acc, 0)

    strip_base = pl.multiple_of(s * STRIP, 8)
    pub_base = pl.multiple_of(s * CB, 8)

    for r in range(R_ROUNDS):
        pub = pltpu.make_async_copy(
            hist_v.at[pl.ds(r * CB, CB)], shared_sp.at[pl.ds(pub_base, CB)],
            sem_r)
        pub.start()
        pub.wait()
        plsc.subcore_barrier()

        reads = [
            pltpu.make_async_copy(
                shared_sp.at[pl.ds(pl.multiple_of(t * CB + s * STRIP, 8),
                                   STRIP)],
                tmp_v.at[pl.ds(t * STRIP, STRIP)], sem_r)
            for t in range(N_SUBCORES)
        ]
        for rd in reads:
            rd.start()
        for rd in reads:
            rd.wait()

        def _sum(i, carry):
            v = tmp_v[pl.ds(i * L, L)]
            for t in range(1, N_SUBCORES):
                v = v + tmp_v[pl.ds(t * STRIP + i * L, L)]
            da = pl.ds(r * STRIP + i * L, L)
            acc_v[da] = acc_v[da] + v
            return carry

        lax.fori_loop(0, STRIP // L, _sum, 0)
        plsc.subcore_barrier()

    def _clip(i, carry):
        d = pl.ds(i * L, L)
        acc_v[d] = jnp.minimum(acc_v[d], 63)
        return carry

    lax.fori_loop(0, SLICE // L, _clip, 0)

    for r in range(R_ROUNDS):

        @pl.when(c == 0)
        def _():
            pltpu.sync_copy(acc_v.at[pl.ds(r * STRIP, STRIP)],
                            deg_in_hbm.at[pl.ds(r * CB + s * STRIP, STRIP)])

        @pl.when(c != 0)
        def _():
            pltpu.sync_copy(acc_v.at[pl.ds(r * STRIP, STRIP)],
                            deg_out_hbm.at[pl.ds(r * CB + s * STRIP, STRIP)])


_sc_bincount = pl.kernel(
    _hist_body,
    out_type=(
        jax.ShapeDtypeStruct((NPAD,), jnp.int32),
        jax.ShapeDtypeStruct((NPAD,), jnp.int32),
    ),
    mesh=plsc.VectorSubcoreMesh(core_axis_name="c", subcore_axis_name="s"),
    compiler_params=pltpu.CompilerParams(needs_layout_passes=False),
    scratch_types=(
        pltpu.VMEM((NPAD,), jnp.int32),
        pltpu.VMEM((2 * CH,), jnp.int32),
        pltpu.VMEM((SLICE,), jnp.int32),
        pltpu.VMEM((N_SUBCORES * STRIP,), jnp.int32),
        pltpu.VMEM_SHARED((N_SUBCORES * CB,), jnp.int32),
        pltpu.SemaphoreType.DMA,
        pltpu.SemaphoreType.DMA,
        pltpu.SemaphoreType.DMA,
    ),
)


GROWS = 3200
GROUP = 800
TABW = 64 * FEAT


def _gath_body(din_hbm, dout_hbm, intab_hbm, outtab_hbm, out_hbm,
               din_v, dout_v, intab_v, outtab_v, stage_v):
    c = lax.axis_index("c")
    s = lax.axis_index("s")
    wid = s * 2 + c
    base = pl.multiple_of(wid * GROWS, 8)

    pltpu.sync_copy(intab_hbm, intab_v)
    pltpu.sync_copy(outtab_hbm, outtab_v)
    pltpu.sync_copy(din_hbm.at[pl.ds(base, GROWS)], din_v)
    pltpu.sync_copy(dout_hbm.at[pl.ds(base, GROWS)], dout_v)

    ngroups = jnp.where(wid == 31, 1, 4)
    lane = lax.iota(jnp.int32, L)

    for g in range(4):

        @pl.when(g < ngroups)
        def _():
            def _q(q, carry):
                off = pl.ds(g * GROUP + q * L, L)
                ib = din_v[off] * FEAT
                ob = dout_v[off] * FEAT
                rows = q * L + lane
                for f in range(FEAT):
                    fv = jnp.full((L,), f, jnp.int32)
                    va = plsc.load_gather(intab_v, [ib + f])
                    vb = plsc.load_gather(outtab_v, [ob + f])
                    plsc.store_scatter(stage_v, [rows, fv], va + vb)
                return carry

            lax.fori_loop(0, GROUP // L, _q, 0)
            pltpu.sync_copy(
                stage_v, out_hbm.at[pl.ds(base + g * GROUP, GROUP), :])


_sc_gather = pl.kernel(
    _gath_body,
    out_type=jax.ShapeDtypeStruct((N_NODES, FEAT), jnp.float32),
    mesh=plsc.VectorSubcoreMesh(core_axis_name="c", subcore_axis_name="s"),
    compiler_params=pltpu.CompilerParams(needs_layout_passes=False),
    scratch_types=(
        pltpu.VMEM((GROWS,), jnp.int32),
        pltpu.VMEM((GROWS,), jnp.int32),
        pltpu.VMEM((TABW,), jnp.float32),
        pltpu.VMEM((TABW,), jnp.float32),
        pltpu.VMEM((GROUP, FEAT), jnp.float32),
    ),
)


@jax.jit
def kernel(in_embed, out_embed, edge_index_list):
    eflat = edge_index_list.astype(jnp.int32).reshape(2 * N_EDGES)
    deg_in, deg_out = _sc_bincount(eflat)
    return _sc_gather(deg_in, deg_out,
                      in_embed.reshape(TABW), out_embed.reshape(TABW))


# speedup vs baseline: 1.4414x; 1.4414x over previous
import jax
import jax.numpy as jnp
from jax import lax
from jax.experimental import pallas as pl
from jax.experimental.pallas import tpu as pltpu
from jax.experimental.pallas import tpu_sc as plsc

N_NODES = 100000
N_EDGES = 3200000
FEAT = 32
NPAD = 102400
N_SUBCORES = 16
SLICE = NPAD // N_SUBCORES
EDGES_PER_TILE = N_EDGES // N_SUBCORES
CH = 4000
NCH = EDGES_PER_TILE // CH
L = 16
R_ROUNDS = 16
CB = NPAD // R_ROUNDS
STRIP = CB // N_SUBCORES
U = 10


def _hist_body(eflat_hbm, deg_in_hbm, deg_out_hbm,
               hist_v, ebuf_v, acc_v, tmp_v, shared_sp,
               sem_a, sem_b, sem_r):
    c = lax.axis_index("c")
    s = lax.axis_index("s")

    zeros = jnp.zeros((L,), jnp.int32)
    ones = jnp.ones((L,), jnp.int32)

    def _zero_hist(i, carry):
        for u in range(8):
            hist_v[pl.ds(i * 8 * L + u * L, L)] = zeros
        return carry

    lax.fori_loop(0, NPAD // (8 * L), _zero_hist, 0)

    base = (1 - c) * N_EDGES + s * EDGES_PER_TILE

    def _edge_copy(k, b):
        sem = sem_a if b == 0 else sem_b
        return pltpu.make_async_copy(
            eflat_hbm.at[pl.ds(base + k * CH, CH)],
            ebuf_v.at[pl.ds(b * CH, CH)], sem)

    _edge_copy(0, 0).start()

    def _chunk_pair(kk, carry):
        for b in range(2):
            k = kk * 2 + b
            _edge_copy(k, b).wait()

            @pl.when(k + 1 < NCH)
            def _():
                _edge_copy(k + 1, 1 - b).start()

            def _scat(j, carry2):
                for u in range(U):
                    idx = ebuf_v[pl.ds(b * CH + j * (U * L) + u * L, L)]
                    plsc.addupdate_scatter(hist_v, [idx], ones)
                return carry2

            lax.fori_loop(0, CH // (U * L), _scat, 0)
        return carry

    lax.fori_loop(0, NCH // 2, _chunk_pair, 0)

    def _zero_acc(i, carry):
        acc_v[pl.ds(i * L, L)] = zeros
        return carry

    lax.fori_loop(0, SLICE // L, _zero_acc, 0)

    strip_base = pl.multiple_of(s * STRIP, 8)
    pub_base = pl.multiple_of(s * CB, 8)

    for r in range(R_ROUNDS):
        pub = pltpu.make_async_copy(
            hist_v.at[pl.ds(r * CB, CB)], shared_sp.at[pl.ds(pub_base, CB)],
            sem_r)
        pub.start()
        pub.wait()
        plsc.subcore_barrier()

        reads = [
            pltpu.make_async_copy(
                shared_sp.at[pl.ds(pl.multiple_of(t * CB + s * STRIP, 8),
                                   STRIP)],
                tmp_v.at[pl.ds(t * STRIP, STRIP)], sem_r)
            for t in range(N_SUBCORES)
        ]
        for rd in reads:
            rd.start()
        for rd in reads:
            rd.wait()

        def _sum(i, carry):
            v = tmp_v[pl.ds(i * L, L)]
            for t in range(1, N_SUBCORES):
                v = v + tmp_v[pl.ds(t * STRIP + i * L, L)]
            da = pl.ds(r * STRIP + i * L, L)
            acc_v[da] = acc_v[da] + v
            return carry

        lax.fori_loop(0, STRIP // L, _sum, 0)
        plsc.subcore_barrier()

    def _clip(i, carry):
        d = pl.ds(i * L, L)
        acc_v[d] = jnp.minimum(acc_v[d], 63)
        return carry

    lax.fori_loop(0, SLICE // L, _clip, 0)

    for r in range(R_ROUNDS):

        @pl.when(c == 0)
        def _():
            pltpu.sync_copy(acc_v.at[pl.ds(r * STRIP, STRIP)],
                            deg_in_hbm.at[pl.ds(r * CB + s * STRIP, STRIP)])

        @pl.when(c != 0)
        def _():
            pltpu.sync_copy(acc_v.at[pl.ds(r * STRIP, STRIP)],
                            deg_out_hbm.at[pl.ds(r * CB + s * STRIP, STRIP)])


_sc_bincount = pl.kernel(
    _hist_body,
    out_type=(
        jax.ShapeDtypeStruct((NPAD,), jnp.int32),
        jax.ShapeDtypeStruct((NPAD,), jnp.int32),
    ),
    mesh=plsc.VectorSubcoreMesh(core_axis_name="c", subcore_axis_name="s"),
    compiler_params=pltpu.CompilerParams(needs_layout_passes=False),
    scratch_types=(
        pltpu.VMEM((NPAD,), jnp.int32),
        pltpu.VMEM((2 * CH,), jnp.int32),
        pltpu.VMEM((SLICE,), jnp.int32),
        pltpu.VMEM((N_SUBCORES * STRIP,), jnp.int32),
        pltpu.VMEM_SHARED((N_SUBCORES * CB,), jnp.int32),
        pltpu.SemaphoreType.DMA,
        pltpu.SemaphoreType.DMA,
        pltpu.SemaphoreType.DMA,
    ),
)


GROWS = 3200
GCH = 128
NGCH = GROWS // GCH


def _tab_body(in_ref, out_ref, tab_ref):
    a = (in_ref[...][:, None, :] + out_ref[...][None, :, :])
    a = a.reshape(64 * 64, FEAT)
    tab_ref[...] = jnp.concatenate(
        [a, jnp.zeros((64 * 64, 128 - FEAT), jnp.float32)], axis=1)


_tc_table = pl.pallas_call(
    _tab_body,
    out_shape=jax.ShapeDtypeStruct((64 * 64, 128), jnp.float32),
)


def _gath_body(din_hbm, dout_hbm, tab_hbm, out_hbm,
               din_v, dout_v, comb_v, rv_a, rv_b, rv_t,
               gsem_a, gsem_b, wsem_a, wsem_b):
    c = lax.axis_index("c")
    s = lax.axis_index("s")
    wid = s * 2 + c
    base = pl.multiple_of(wid * GROWS, 8)

    pltpu.sync_copy(din_hbm.at[pl.ds(base, GROWS)], din_v)
    pltpu.sync_copy(dout_hbm.at[pl.ds(base, GROWS)], dout_v)

    def _comb(q, carry):
        d = pl.ds(q * L, L)
        comb_v[d] = din_v[d] * 64 + dout_v[d]
        return carry

    lax.fori_loop(0, GROWS // L, _comb, 0)

    nch = jnp.where(wid == 31, GROWS // 4 // GCH, NGCH)

    def _gather(k, b):
        gsem = gsem_a if b == 0 else gsem_b
        rv = rv_a if b == 0 else rv_b
        return pltpu.make_async_copy(
            tab_hbm.at[comb_v.at[pl.ds(k * GCH, GCH)]], rv, gsem)

    def _write(k, b):
        wsem = wsem_a if b == 0 else wsem_b
        rv = rv_a if b == 0 else rv_b
        return pltpu.make_async_copy(
            rv, out_hbm.at[pl.ds(base + k * GCH, GCH), :], wsem)

    _gather(0, 0).start()

    def _pair(j, carry):
        for b in range(2):
            k = j * 2 + b

            @pl.when(k < nch)
            def _():
                _gather(k, b).wait()

                @pl.when(k + 1 < nch)
                def _():
                    @pl.when(k >= 1)
                    def _():
                        _write(k - 1, 1 - b).wait()

                    _gather(k + 1, 1 - b).start()

                _write(k, b).start()

        return carry

    lax.fori_loop(0, (NGCH + 1) // 2, _pair, 0)

    @pl.when(nch % 2 == 1)
    def _():
        _write(nch - 2, 1).wait()
        _write(nch - 1, 0).wait()

    @pl.when(nch % 2 == 0)
    def _():
        _write(nch - 2, 0).wait()
        _write(nch - 1, 1).wait()

    @pl.when(wid == 31)
    def _():
        gt = pltpu.make_async_copy(
            tab_hbm.at[comb_v.at[pl.ds(6 * GCH, 32)]], rv_t, gsem_a)
        gt.start()
        gt.wait()
        wt = pltpu.make_async_copy(
            rv_t, out_hbm.at[pl.ds(base + 6 * GCH, 32), :], wsem_a)
        wt.start()
        wt.wait()


_sc_gather = pl.kernel(
    _gath_body,
    out_type=jax.ShapeDtypeStruct((N_NODES, 128), jnp.float32),
    mesh=plsc.VectorSubcoreMesh(core_axis_name="c", subcore_axis_name="s"),
    compiler_params=pltpu.CompilerParams(needs_layout_passes=False),
    scratch_types=(
        pltpu.VMEM((GROWS,), jnp.int32),
        pltpu.VMEM((GROWS,), jnp.int32),
        pltpu.VMEM((GROWS,), jnp.int32),
        pltpu.VMEM((GCH, 128), jnp.float32),
        pltpu.VMEM((GCH, 128), jnp.float32),
        pltpu.VMEM((32, 128), jnp.float32),
        pltpu.SemaphoreType.DMA,
        pltpu.SemaphoreType.DMA,
        pltpu.SemaphoreType.DMA,
        pltpu.SemaphoreType.DMA,
    ),
)


@jax.jit
def kernel(in_embed, out_embed, edge_index_list):
    eflat = edge_index_list.astype(jnp.int32).reshape(2 * N_EDGES)
    deg_in, deg_out = _sc_bincount(eflat)
    table = _tc_table(in_embed, out_embed)
    return _sc_gather(deg_in, deg_out, table)[:, :FEAT]


# speedup vs baseline: 1.5705x; 1.0896x over previous
import jax
import jax.numpy as jnp
from jax import lax
from jax.experimental import pallas as pl
from jax.experimental.pallas import tpu as pltpu
from jax.experimental.pallas import tpu_sc as plsc

N_NODES = 100000
N_EDGES = 3200000
FEAT = 32
NPAD = 102400
N_SUBCORES = 16
SLICE = NPAD // N_SUBCORES
EDGES_PER_TILE = N_EDGES // N_SUBCORES
CH = 4000
NCH = EDGES_PER_TILE // CH
L = 16
R_ROUNDS = 16
CB = NPAD // R_ROUNDS
STRIP = CB // N_SUBCORES
U = 10


def _hist_body(eflat_hbm, deg_in_hbm, deg_out_hbm,
               hist_v, ebuf_v, acc_v, tmp_v, shared_sp,
               sem_a, sem_b, sem_r):
    c = lax.axis_index("c")
    s = lax.axis_index("s")

    zeros = jnp.zeros((L,), jnp.int32)
    ones = jnp.ones((L,), jnp.int32)

    def _zero_hist(i, carry):
        for u in range(8):
            hist_v[pl.ds(i * 8 * L + u * L, L)] = zeros
        return carry

    lax.fori_loop(0, NPAD // (8 * L), _zero_hist, 0)

    base = (1 - c) * N_EDGES + s * EDGES_PER_TILE

    def _edge_copy(k, b):
        sem = sem_a if b == 0 else sem_b
        return pltpu.make_async_copy(
            eflat_hbm.at[pl.ds(base + k * CH, CH)],
            ebuf_v.at[pl.ds(b * CH, CH)], sem)

    _edge_copy(0, 0).start()

    def _chunk_pair(kk, carry):
        for b in range(2):
            k = kk * 2 + b
            _edge_copy(k, b).wait()

            @pl.when(k + 1 < NCH)
            def _():
                _edge_copy(k + 1, 1 - b).start()

            def _scat(j, carry2):
                for u in range(U):
                    idx = ebuf_v[pl.ds(b * CH + j * (U * L) + u * L, L)]
                    plsc.addupdate_scatter(hist_v, [idx], ones)
                return carry2

            lax.fori_loop(0, CH // (U * L), _scat, 0)
        return carry

    lax.fori_loop(0, NCH // 2, _chunk_pair, 0)

    def _zero_acc(i, carry):
        acc_v[pl.ds(i * L, L)] = zeros
        return carry

    lax.fori_loop(0, SLICE // L, _zero_acc, 0)

    strip_base = pl.multiple_of(s * STRIP, 8)
    pub_base = pl.multiple_of(s * CB, 8)

    for r in range(R_ROUNDS):
        pub = pltpu.make_async_copy(
            hist_v.at[pl.ds(r * CB, CB)], shared_sp.at[pl.ds(pub_base, CB)],
            sem_r)
        pub.start()
        pub.wait()
        plsc.subcore_barrier()

        reads = [
            pltpu.make_async_copy(
                shared_sp.at[pl.ds(pl.multiple_of(t * CB + s * STRIP, 8),
                                   STRIP)],
                tmp_v.at[pl.ds(t * STRIP, STRIP)], sem_r)
            for t in range(N_SUBCORES)
        ]
        for rd in reads:
            rd.start()
        for rd in reads:
            rd.wait()

        def _sum(i, carry):
            v = tmp_v[pl.ds(i * L, L)]
            for t in range(1, N_SUBCORES):
                v = v + tmp_v[pl.ds(t * STRIP + i * L, L)]
            da = pl.ds(r * STRIP + i * L, L)
            acc_v[da] = acc_v[da] + v
            return carry

        lax.fori_loop(0, STRIP // L, _sum, 0)
        plsc.subcore_barrier()

    def _clip(i, carry):
        d = pl.ds(i * L, L)
        acc_v[d] = jnp.minimum(acc_v[d], 63)
        return carry

    lax.fori_loop(0, SLICE // L, _clip, 0)

    for r in range(R_ROUNDS):

        @pl.when(c == 0)
        def _():
            pltpu.sync_copy(acc_v.at[pl.ds(r * STRIP, STRIP)],
                            deg_in_hbm.at[pl.ds(r * CB + s * STRIP, STRIP)])

        @pl.when(c != 0)
        def _():
            pltpu.sync_copy(acc_v.at[pl.ds(r * STRIP, STRIP)],
                            deg_out_hbm.at[pl.ds(r * CB + s * STRIP, STRIP)])


_sc_bincount = pl.kernel(
    _hist_body,
    out_type=(
        jax.ShapeDtypeStruct((NPAD,), jnp.int32),
        jax.ShapeDtypeStruct((NPAD,), jnp.int32),
    ),
    mesh=plsc.VectorSubcoreMesh(core_axis_name="c", subcore_axis_name="s"),
    compiler_params=pltpu.CompilerParams(needs_layout_passes=False),
    scratch_types=(
        pltpu.VMEM((NPAD,), jnp.int32),
        pltpu.VMEM((2 * CH,), jnp.int32),
        pltpu.VMEM((SLICE,), jnp.int32),
        pltpu.VMEM((N_SUBCORES * STRIP,), jnp.int32),
        pltpu.VMEM_SHARED((N_SUBCORES * CB,), jnp.int32),
        pltpu.SemaphoreType.DMA,
        pltpu.SemaphoreType.DMA,
        pltpu.SemaphoreType.DMA,
    ),
)


BLK = 2048


def _gather_body(din_ref, dout_ref, tab_ref, out_ref):
    di = din_ref[...]
    do = dout_ref[...]
    col = lax.broadcasted_iota(jnp.int32, (BLK, 2 * 64), 1)
    target = jnp.where(col < 64, di[:, None], do[:, None] + 64)
    oh = jnp.where(target == col, jnp.float32(1), jnp.float32(0))
    out_ref[...] = jax.lax.dot(
        oh, tab_ref[...], preferred_element_type=jnp.float32)


_tc_gather = pl.pallas_call(
    _gather_body,
    grid=(NPAD // BLK,),
    in_specs=[
        pl.BlockSpec((BLK,), lambda i: (i,)),
        pl.BlockSpec((BLK,), lambda i: (i,)),
        pl.BlockSpec((2 * 64, FEAT), lambda i: (0, 0)),
    ],
    out_specs=pl.BlockSpec((BLK, FEAT), lambda i: (i, 0)),
    out_shape=jax.ShapeDtypeStruct((NPAD, FEAT), jnp.float32),
)


@jax.jit
def kernel(in_embed, out_embed, edge_index_list):
    eflat = edge_index_list.astype(jnp.int32).reshape(2 * N_EDGES)
    deg_in, deg_out = _sc_bincount(eflat)
    table = jnp.concatenate([in_embed, out_embed], axis=0)
    return _tc_gather(deg_in, deg_out, table)[:N_NODES]
